# baseline (speedup 1.0000x reference)
import functools

import jax
import jax.numpy as jnp
from jax import lax
from jax.experimental import pallas as pl
from jax.experimental.pallas import tpu as pltpu
from jax.experimental.pallas import tpu_sc as plsc

N_S = 10000
N_T = 10000
D_FEAT = 64
NUM_CLASSES = 22
C_PAD = 128
TEMP = 2.0

BS = 1000
N_SB = N_S // BS
NT_PAD = 10240
TT = 2048

B_PAD = 10240
NW = 32
B_PER_W = B_PAD // NW


def _log_softmax(x):
    m = jnp.max(x, axis=1, keepdims=True)
    y = x - m
    return y - jnp.log(jnp.sum(jnp.exp(y), axis=1, keepdims=True))


def _body_a(s_aug_ref, t_aug_t_ref, s_feat_ref, t_feat_ref, w_ref, b_ref,
            wt_ref, bt_ref, seg_ref, idx_ref, ce_ref, sl_ref, t_logits_ref):
    pid = pl.program_id(0)

    @pl.when(pid == 0)
    def _compute_teacher_logits():
        t_logits_ref[...] = (
            jnp.dot(t_feat_ref[...], wt_ref[...],
                    preferred_element_type=jnp.float32) + bt_ref[...])

    s_aug = s_aug_ref[...]

    def t_tile(i, carry):
        best, bidx = carry
        t_blk = t_aug_t_ref[:, pl.ds(i * TT, TT)]
        sc = jnp.dot(s_aug, t_blk, preferred_element_type=jnp.float32,
                     precision=jax.lax.Precision.HIGHEST)
        m = jnp.max(sc, axis=1, keepdims=True)
        col = jax.lax.broadcasted_iota(jnp.int32, (BS, TT), 1) + i * TT
        li = jnp.min(jnp.where(sc == m, col, jnp.int32(2**30)),
                     axis=1, keepdims=True)
        upd = m > best
        return jnp.where(upd, m, best), jnp.where(upd, li, bidx)

    best0 = jnp.full((BS, 1), -jnp.inf, dtype=jnp.float32)
    bidx0 = jnp.zeros((BS, 1), dtype=jnp.int32)
    _, bidx = jax.lax.fori_loop(0, NT_PAD // TT, t_tile, (best0, bidx0))
    idx_ref[...] = bidx[:, 0].reshape(1, 1, BS)

    sl = (jnp.dot(s_feat_ref[...], w_ref[...],
                  preferred_element_type=jnp.float32) + b_ref[...])
    sl_ref[...] = sl
    logp = _log_softmax(sl)
    seg = seg_ref[0, 0, :]
    cls = jax.lax.broadcasted_iota(jnp.int32, (BS, NUM_CLASSES), 1)
    seg_oh = cls == seg[:, None]
    ce_ref[...] = jnp.broadcast_to(-jnp.sum(jnp.where(seg_oh, logp, 0.0)),
                                   (1, 1, 128))


@functools.cache
def _make_sc_gather():
    @functools.partial(
        pl.kernel,
        mesh=plsc.VectorSubcoreMesh(core_axis_name="c", subcore_axis_name="s"),
        out_type=jax.ShapeDtypeStruct((B_PAD, C_PAD), jnp.float32),
        scratch_types=[
            pltpu.VMEM((B_PER_W,), jnp.int32),
            pltpu.VMEM((B_PER_W, C_PAD), jnp.float32),
            pltpu.SemaphoreType.DMA,
        ],
    )
    def _sc_gather(table_hbm, idx_hbm, out_hbm, idx_v, rows_v, sem):
        wid = lax.axis_index("s") * 2 + lax.axis_index("c")
        base = wid * B_PER_W
        pltpu.sync_copy(idx_hbm.at[pl.ds(base, B_PER_W)], idx_v)
        pltpu.async_copy(table_hbm.at[idx_v], rows_v, sem).wait()
        pltpu.sync_copy(rows_v, out_hbm.at[pl.ds(base, B_PER_W)])

    return _sc_gather


def _body_c(matched_ref, sl_ref, kl_ref):
    mt = matched_ref[...][:, :NUM_CLASSES] / TEMP
    slp = _log_softmax(sl_ref[...] / TEMP)
    tlp = _log_softmax(mt)
    tp = jnp.exp(tlp)
    kl_ref[...] = jnp.broadcast_to(jnp.sum(tp * (tlp - slp)), (1, 128))


@jax.jit
def kernel(s_feat, t_feat, student_coords, teacher_coords, W, b, Wt, bt,
           segment):
    s_aug = jnp.concatenate(
        [student_coords, jnp.ones((N_S, 1), jnp.float32)], axis=1)
    t2 = jnp.sum(teacher_coords * teacher_coords, axis=1)
    t_aug_t = jnp.concatenate([teacher_coords.T, (-0.5 * t2)[None, :]], axis=0)
    pad = jnp.zeros((4, NT_PAD - N_T), jnp.float32).at[3, :].set(-1e30)
    t_aug_t = jnp.concatenate([t_aug_t, pad], axis=1)

    seg3 = segment.astype(jnp.int32).reshape(N_SB, 1, BS)
    b2 = b.reshape(1, NUM_CLASSES)
    wt_p = jnp.pad(Wt, ((0, 0), (0, C_PAD - NUM_CLASSES)))
    bt_p = jnp.pad(bt, (0, C_PAD - NUM_CLASSES)).reshape(1, C_PAD)

    idx3, ce_part, s_logits, t_logits = pl.pallas_call(
        _body_a,
        grid=(N_SB,),
        in_specs=[
            pl.BlockSpec((BS, 4), lambda i: (i, 0)),
            pl.BlockSpec((4, NT_PAD), lambda i: (0, 0)),
            pl.BlockSpec((BS, D_FEAT), lambda i: (i, 0)),
            pl.BlockSpec((N_T, D_FEAT), lambda i: (0, 0)),
            pl.BlockSpec((D_FEAT, NUM_CLASSES), lambda i: (0, 0)),
            pl.BlockSpec((1, NUM_CLASSES), lambda i: (0, 0)),
            pl.BlockSpec((D_FEAT, C_PAD), lambda i: (0, 0)),
            pl.BlockSpec((1, C_PAD), lambda i: (0, 0)),
            pl.BlockSpec((1, 1, BS), lambda i: (i, 0, 0)),
        ],
        out_specs=[
            pl.BlockSpec((1, 1, BS), lambda i: (i, 0, 0)),
            pl.BlockSpec((1, 1, 128), lambda i: (i, 0, 0)),
            pl.BlockSpec((BS, NUM_CLASSES), lambda i: (i, 0)),
            pl.BlockSpec((N_T, C_PAD), lambda i: (0, 0)),
        ],
        out_shape=[
            jax.ShapeDtypeStruct((N_SB, 1, BS), jnp.int32),
            jax.ShapeDtypeStruct((N_SB, 1, 128), jnp.float32),
            jax.ShapeDtypeStruct((N_S, NUM_CLASSES), jnp.float32),
            jax.ShapeDtypeStruct((N_T, C_PAD), jnp.float32),
        ],
    )(s_aug, t_aug_t, s_feat, t_feat, W, b2, wt_p, bt_p, seg3)

    idx_pad = jnp.pad(idx3.reshape(N_S), (0, B_PAD - N_S))
    matched = _make_sc_gather()(t_logits, idx_pad)

    kl_part = pl.pallas_call(
        _body_c,
        grid=(1,),
        in_specs=[
            pl.BlockSpec((N_S, C_PAD), lambda i: (0, 0)),
            pl.BlockSpec((N_S, NUM_CLASSES), lambda i: (0, 0)),
        ],
        out_specs=pl.BlockSpec((1, 128), lambda i: (0, 0)),
        out_shape=jax.ShapeDtypeStruct((1, 128), jnp.float32),
    )(matched, s_logits)

    seg_loss = jnp.sum(ce_part[:, 0, 0]) / N_S
    kl1 = kl_part[0, 0] / N_S * (TEMP ** 2)
    kl_loss = 0.2 * kl1
    total_loss = seg_loss + kl_loss
    return (total_loss, seg_loss, kl_loss)


# speedup vs baseline: 1.0809x; 1.0809x over previous
import functools

import jax
import jax.numpy as jnp
from jax import lax
from jax.experimental import pallas as pl
from jax.experimental.pallas import tpu as pltpu
from jax.experimental.pallas import tpu_sc as plsc

N_S = 10000
N_T = 10000
D_FEAT = 64
NUM_CLASSES = 22
C_PAD = 128
TEMP = 2.0

BS = 1000
N_SB = N_S // BS
NT_PAD = 10240
TT = 2048

B_PAD = 10240
NW = 32
B_PER_W = B_PAD // NW


def _log_softmax(x):
    m = jnp.max(x, axis=1, keepdims=True)
    y = x - m
    return y - jnp.log(jnp.sum(jnp.exp(y), axis=1, keepdims=True))


def _body_a(s_aug_ref, t_aug_t_ref, s_feat_ref, t_feat_ref, w_ref, b_ref,
            wt_ref, bt_ref, seg_ref, idx_ref, ce_ref, sl_ref, t_logits_ref):
    pid = pl.program_id(0)

    @pl.when(pid == 0)
    def _compute_teacher_logits():
        t_logits_ref[...] = (
            jnp.dot(t_feat_ref[...], wt_ref[...],
                    preferred_element_type=jnp.float32) + bt_ref[...])

    s_aug = s_aug_ref[...]

    best = jnp.full((BS, 1), -jnp.inf, dtype=jnp.float32)
    bidx = jnp.zeros((BS, 1), dtype=jnp.int32)
    for i in range(NT_PAD // TT):
        t_blk = t_aug_t_ref[:, i * TT:(i + 1) * TT]
        sc = jnp.dot(s_aug, t_blk, preferred_element_type=jnp.float32,
                     precision=jax.lax.Precision.HIGHEST)
        m = jnp.max(sc, axis=1, keepdims=True)
        col = jax.lax.broadcasted_iota(jnp.int32, (BS, TT), 1) + i * TT
        li = jnp.min(jnp.where(sc == m, col, jnp.int32(2**30)),
                     axis=1, keepdims=True)
        upd = m > best
        best = jnp.where(upd, m, best)
        bidx = jnp.where(upd, li, bidx)
    idx_ref[...] = bidx[:, 0].reshape(1, 1, BS)

    sl = (jnp.dot(s_feat_ref[...], w_ref[...],
                  preferred_element_type=jnp.float32) + b_ref[...])
    sl_ref[...] = sl
    logp = _log_softmax(sl)
    seg = seg_ref[0, 0, :]
    cls = jax.lax.broadcasted_iota(jnp.int32, (BS, NUM_CLASSES), 1)
    seg_oh = cls == seg[:, None]
    ce_ref[...] = jnp.broadcast_to(-jnp.sum(jnp.where(seg_oh, logp, 0.0)),
                                   (1, 1, 128))


@functools.cache
def _make_sc_gather():
    @functools.partial(
        pl.kernel,
        mesh=plsc.VectorSubcoreMesh(core_axis_name="c", subcore_axis_name="s"),
        out_type=jax.ShapeDtypeStruct((B_PAD, C_PAD), jnp.float32),
        scratch_types=[
            pltpu.VMEM((B_PER_W,), jnp.int32),
            pltpu.VMEM((B_PER_W, C_PAD), jnp.float32),
            pltpu.SemaphoreType.DMA,
        ],
    )
    def _sc_gather(table_hbm, idx_hbm, out_hbm, idx_v, rows_v, sem):
        wid = lax.axis_index("s") * 2 + lax.axis_index("c")
        base = wid * B_PER_W
        pltpu.sync_copy(idx_hbm.at[pl.ds(base, B_PER_W)], idx_v)
        pltpu.async_copy(table_hbm.at[idx_v], rows_v, sem).wait()
        pltpu.sync_copy(rows_v, out_hbm.at[pl.ds(base, B_PER_W)])

    return _sc_gather


def _body_c(matched_ref, sl_ref, kl_ref):
    mt = matched_ref[...][:, :NUM_CLASSES] / TEMP
    slp = _log_softmax(sl_ref[...] / TEMP)
    tlp = _log_softmax(mt)
    tp = jnp.exp(tlp)
    kl_ref[...] = jnp.broadcast_to(jnp.sum(tp * (tlp - slp)), (1, 128))


@jax.jit
def kernel(s_feat, t_feat, student_coords, teacher_coords, W, b, Wt, bt,
           segment):
    s_aug = jnp.concatenate(
        [student_coords, jnp.ones((N_S, 1), jnp.float32)], axis=1)
    t2 = jnp.sum(teacher_coords * teacher_coords, axis=1)
    t_aug_t = jnp.concatenate([teacher_coords.T, (-0.5 * t2)[None, :]], axis=0)
    pad = jnp.zeros((4, NT_PAD - N_T), jnp.float32).at[3, :].set(-1e30)
    t_aug_t = jnp.concatenate([t_aug_t, pad], axis=1)

    seg3 = segment.astype(jnp.int32).reshape(N_SB, 1, BS)
    b2 = b.reshape(1, NUM_CLASSES)
    wt_p = jnp.pad(Wt, ((0, 0), (0, C_PAD - NUM_CLASSES)))
    bt_p = jnp.pad(bt, (0, C_PAD - NUM_CLASSES)).reshape(1, C_PAD)

    idx3, ce_part, s_logits, t_logits = pl.pallas_call(
        _body_a,
        grid=(N_SB,),
        in_specs=[
            pl.BlockSpec((BS, 4), lambda i: (i, 0)),
            pl.BlockSpec((4, NT_PAD), lambda i: (0, 0)),
            pl.BlockSpec((BS, D_FEAT), lambda i: (i, 0)),
            pl.BlockSpec((N_T, D_FEAT), lambda i: (0, 0)),
            pl.BlockSpec((D_FEAT, NUM_CLASSES), lambda i: (0, 0)),
            pl.BlockSpec((1, NUM_CLASSES), lambda i: (0, 0)),
            pl.BlockSpec((D_FEAT, C_PAD), lambda i: (0, 0)),
            pl.BlockSpec((1, C_PAD), lambda i: (0, 0)),
            pl.BlockSpec((1, 1, BS), lambda i: (i, 0, 0)),
        ],
        out_specs=[
            pl.BlockSpec((1, 1, BS), lambda i: (i, 0, 0)),
            pl.BlockSpec((1, 1, 128), lambda i: (i, 0, 0)),
            pl.BlockSpec((BS, NUM_CLASSES), lambda i: (i, 0)),
            pl.BlockSpec((N_T, C_PAD), lambda i: (0, 0)),
        ],
        out_shape=[
            jax.ShapeDtypeStruct((N_SB, 1, BS), jnp.int32),
            jax.ShapeDtypeStruct((N_SB, 1, 128), jnp.float32),
            jax.ShapeDtypeStruct((N_S, NUM_CLASSES), jnp.float32),
            jax.ShapeDtypeStruct((N_T, C_PAD), jnp.float32),
        ],
    )(s_aug, t_aug_t, s_feat, t_feat, W, b2, wt_p, bt_p, seg3)

    idx_pad = jnp.pad(idx3.reshape(N_S), (0, B_PAD - N_S))
    matched = _make_sc_gather()(t_logits, idx_pad)

    kl_part = pl.pallas_call(
        _body_c,
        grid=(1,),
        in_specs=[
            pl.BlockSpec((N_S, C_PAD), lambda i: (0, 0)),
            pl.BlockSpec((N_S, NUM_CLASSES), lambda i: (0, 0)),
        ],
        out_specs=pl.BlockSpec((1, 128), lambda i: (0, 0)),
        out_shape=jax.ShapeDtypeStruct((1, 128), jnp.float32),
    )(matched, s_logits)

    seg_loss = jnp.sum(ce_part[:, 0, 0]) / N_S
    kl1 = kl_part[0, 0] / N_S * (TEMP ** 2)
    kl_loss = 0.2 * kl1
    total_loss = seg_loss + kl_loss
    return (total_loss, seg_loss, kl_loss)


# speedup vs baseline: 2.4167x; 2.2358x over previous
import functools

import jax
import jax.numpy as jnp
from jax import lax
from jax.experimental import pallas as pl
from jax.experimental.pallas import tpu as pltpu
from jax.experimental.pallas import tpu_sc as plsc

N_S = 10000
N_T = 10000
D_FEAT = 64
NUM_CLASSES = 22
C_PAD = 128
TEMP = 2.0

KSPLIT = 32
BS = 1000
N_SB = N_S // BS
NT_PAD = 10240
TT = 2048

B_PAD = 10240
NW = 32
B_PER_W = B_PAD // NW


def _log_softmax(x):
    m = jnp.max(x, axis=1, keepdims=True)
    y = x - m
    return y - jnp.log(jnp.sum(jnp.exp(y), axis=1, keepdims=True))


def _body_a(s_aug_ref, t_aug_t_ref, s_feat_ref, t_feat_ref, w_ref, b_ref,
            wt_ref, bt_ref, seg_ref, idx_ref, ce_ref, sl_ref, t_logits_ref):
    pid = pl.program_id(0)

    @pl.when(pid == 0)
    def _compute_teacher_logits():
        t_logits_ref[...] = (
            jnp.dot(t_feat_ref[...], wt_ref[...],
                    preferred_element_type=jnp.float32) + bt_ref[...])

    s_aug = s_aug_ref[...]

    best = jnp.full((BS, 1), -jnp.inf, dtype=jnp.float32)
    bidx = jnp.zeros((BS, 1), dtype=jnp.int32)
    for i in range(NT_PAD // TT):
        t_blk = t_aug_t_ref[:, i * TT:(i + 1) * TT]
        sc = jnp.dot(s_aug, t_blk, preferred_element_type=jnp.float32)
        m = jnp.max(sc, axis=1, keepdims=True)
        col = jax.lax.broadcasted_iota(jnp.int32, (BS, TT), 1) + i * TT
        li = jnp.min(jnp.where(sc == m, col, jnp.int32(2**30)),
                     axis=1, keepdims=True)
        upd = m > best
        best = jnp.where(upd, m, best)
        bidx = jnp.where(upd, li, bidx)
    idx_ref[...] = bidx[:, 0].reshape(1, 1, BS)

    sl = (jnp.dot(s_feat_ref[...], w_ref[...],
                  preferred_element_type=jnp.float32) + b_ref[...])
    sl_ref[...] = sl
    logp = _log_softmax(sl)
    seg = seg_ref[0, 0, :]
    cls = jax.lax.broadcasted_iota(jnp.int32, (BS, NUM_CLASSES), 1)
    seg_oh = cls == seg[:, None]
    ce_ref[...] = jnp.broadcast_to(-jnp.sum(jnp.where(seg_oh, logp, 0.0)),
                                   (1, 1, 128))


@functools.cache
def _make_sc_gather():
    @functools.partial(
        pl.kernel,
        mesh=plsc.VectorSubcoreMesh(core_axis_name="c", subcore_axis_name="s"),
        out_type=jax.ShapeDtypeStruct((B_PAD, C_PAD), jnp.float32),
        scratch_types=[
            pltpu.VMEM((B_PER_W,), jnp.int32),
            pltpu.VMEM((B_PER_W, C_PAD), jnp.float32),
            pltpu.SemaphoreType.DMA,
        ],
    )
    def _sc_gather(table_hbm, idx_hbm, out_hbm, idx_v, rows_v, sem):
        wid = lax.axis_index("s") * 2 + lax.axis_index("c")
        base = wid * B_PER_W
        pltpu.sync_copy(idx_hbm.at[pl.ds(base, B_PER_W)], idx_v)
        pltpu.async_copy(table_hbm.at[idx_v], rows_v, sem).wait()
        pltpu.sync_copy(rows_v, out_hbm.at[pl.ds(base, B_PER_W)])

    return _sc_gather


def _body_c(matched_ref, sl_ref, kl_ref):
    mt = matched_ref[...][:, :NUM_CLASSES] / TEMP
    slp = _log_softmax(sl_ref[...] / TEMP)
    tlp = _log_softmax(mt)
    tp = jnp.exp(tlp)
    kl_ref[...] = jnp.broadcast_to(jnp.sum(tp * (tlp - slp)), (1, 128))


@jax.jit
def kernel(s_feat, t_feat, student_coords, teacher_coords, W, b, Wt, bt,
           segment):
    def split3(x):
        h1 = x.astype(jnp.bfloat16).astype(jnp.float32)
        r1 = x - h1
        h2 = r1.astype(jnp.bfloat16).astype(jnp.float32)
        h3 = (r1 - h2).astype(jnp.bfloat16).astype(jnp.float32)
        return h1, h2, h3

    s_ctr = student_coords - 5.0
    t_ctr = teacher_coords - 5.0

    t2 = jnp.sum(t_ctr * t_ctr, axis=1)
    neg_half_t2 = jnp.concatenate(
        [-0.5 * t2, jnp.full((NT_PAD - N_T,), -1e30, jnp.float32)])
    t_coords_pad = jnp.concatenate(
        [t_ctr, jnp.zeros((NT_PAD - N_T, 3), jnp.float32)], axis=0)

    s_sp = split3(s_ctr)
    t_sp = split3(t_coords_pad)
    b_sp = split3(neg_half_t2)

    lhs_cols, rhs_rows = [], []
    ones = jnp.ones((N_S,), jnp.float32)
    for c in range(3):
        for i in range(3):
            for j in range(3):
                lhs_cols.append(s_sp[i][:, c])
                rhs_rows.append(t_sp[j][:, c])
    for k in range(3):
        lhs_cols.append(ones)
        rhs_rows.append(b_sp[k])
    zs = jnp.zeros((N_S,), jnp.float32)
    zt = jnp.zeros((NT_PAD,), jnp.float32)
    while len(lhs_cols) < KSPLIT:
        lhs_cols.append(zs)
        rhs_rows.append(zt)
    s_aug = jnp.stack(lhs_cols, axis=1).astype(jnp.bfloat16)
    t_aug_t = jnp.stack(rhs_rows, axis=0).astype(jnp.bfloat16)

    seg3 = segment.astype(jnp.int32).reshape(N_SB, 1, BS)
    b2 = b.reshape(1, NUM_CLASSES)
    wt_p = jnp.pad(Wt, ((0, 0), (0, C_PAD - NUM_CLASSES)))
    bt_p = jnp.pad(bt, (0, C_PAD - NUM_CLASSES)).reshape(1, C_PAD)

    idx3, ce_part, s_logits, t_logits = pl.pallas_call(
        _body_a,
        grid=(N_SB,),
        in_specs=[
            pl.BlockSpec((BS, KSPLIT), lambda i: (i, 0)),
            pl.BlockSpec((KSPLIT, NT_PAD), lambda i: (0, 0)),
            pl.BlockSpec((BS, D_FEAT), lambda i: (i, 0)),
            pl.BlockSpec((N_T, D_FEAT), lambda i: (0, 0)),
            pl.BlockSpec((D_FEAT, NUM_CLASSES), lambda i: (0, 0)),
            pl.BlockSpec((1, NUM_CLASSES), lambda i: (0, 0)),
            pl.BlockSpec((D_FEAT, C_PAD), lambda i: (0, 0)),
            pl.BlockSpec((1, C_PAD), lambda i: (0, 0)),
            pl.BlockSpec((1, 1, BS), lambda i: (i, 0, 0)),
        ],
        out_specs=[
            pl.BlockSpec((1, 1, BS), lambda i: (i, 0, 0)),
            pl.BlockSpec((1, 1, 128), lambda i: (i, 0, 0)),
            pl.BlockSpec((BS, NUM_CLASSES), lambda i: (i, 0)),
            pl.BlockSpec((N_T, C_PAD), lambda i: (0, 0)),
        ],
        out_shape=[
            jax.ShapeDtypeStruct((N_SB, 1, BS), jnp.int32),
            jax.ShapeDtypeStruct((N_SB, 1, 128), jnp.float32),
            jax.ShapeDtypeStruct((N_S, NUM_CLASSES), jnp.float32),
            jax.ShapeDtypeStruct((N_T, C_PAD), jnp.float32),
        ],
    )(s_aug, t_aug_t, s_feat, t_feat, W, b2, wt_p, bt_p, seg3)

    idx_pad = jnp.pad(idx3.reshape(N_S), (0, B_PAD - N_S))
    matched = _make_sc_gather()(t_logits, idx_pad)

    kl_part = pl.pallas_call(
        _body_c,
        grid=(1,),
        in_specs=[
            pl.BlockSpec((N_S, C_PAD), lambda i: (0, 0)),
            pl.BlockSpec((N_S, NUM_CLASSES), lambda i: (0, 0)),
        ],
        out_specs=pl.BlockSpec((1, 128), lambda i: (0, 0)),
        out_shape=jax.ShapeDtypeStruct((1, 128), jnp.float32),
    )(matched, s_logits)

    seg_loss = jnp.sum(ce_part[:, 0, 0]) / N_S
    kl1 = kl_part[0, 0] / N_S * (TEMP ** 2)
    kl_loss = 0.2 * kl1
    total_loss = seg_loss + kl_loss
    return (total_loss, seg_loss, kl_loss)
